# Initial kernel scaffold; baseline (speedup 1.0000x reference)
import jax
import jax.numpy as jnp
from jax.experimental import pallas as pl


def kernel(spatial_pos, W):
    raise NotImplementedError("write your pallas kernel here")


# speedup vs baseline: 7.1338x; 7.1338x over previous
import functools

import jax
import jax.numpy as jnp
from jax import lax
from jax.experimental import pallas as pl
from jax.experimental.pallas import tpu as pltpu
from jax.experimental.pallas import tpu_sc as plsc

B, N, EMB = 8, 512, 16
TOTAL = B * N * N
_NC, _NS = 2, 16
NW = _NC * _NS
PER_W = TOTAL // NW
CHUNK = 512
STEPS = PER_W // CHUNK

_mesh = plsc.VectorSubcoreMesh(core_axis_name="c", subcore_axis_name="s")


@functools.partial(
    pl.kernel,
    mesh=_mesh,
    out_type=jax.ShapeDtypeStruct((TOTAL, EMB), jnp.float32),
    scratch_types=[
        pltpu.VMEM((CHUNK,), jnp.int32),
        pltpu.VMEM((CHUNK, EMB), jnp.float32),
        pltpu.SemaphoreType.DMA,
    ],
    compiler_params=pltpu.CompilerParams(use_tc_tiling_on_sc=False),
)
def _gather_kernel(table_hbm, idx_hbm, out_hbm, idx_v, rows_v, sem):
    wid = lax.axis_index("s") * _NC + lax.axis_index("c")
    base = wid * PER_W

    def step(i, carry):
        start = base + i * CHUNK
        pltpu.sync_copy(idx_hbm.at[pl.ds(start, CHUNK)], idx_v)
        pltpu.async_copy(table_hbm.at[idx_v], rows_v, sem).wait()
        pltpu.sync_copy(rows_v, out_hbm.at[pl.ds(start, CHUNK)])
        return carry

    lax.fori_loop(0, STEPS, step, 0)


def kernel(spatial_pos, W):
    idx = spatial_pos.reshape(TOTAL).astype(jnp.int32)
    out = _gather_kernel(W, idx)
    return out.reshape(B, N, N, EMB)


# speedup vs baseline: 7.1455x; 1.0016x over previous
import functools

import jax
import jax.numpy as jnp
from jax import lax
from jax.experimental import pallas as pl
from jax.experimental.pallas import tpu as pltpu
from jax.experimental.pallas import tpu_sc as plsc

B, N, EMB = 8, 512, 16
PAIRS = B * N
_NC, _NS = 2, 16
NW = _NC * _NS
PER_W = PAIRS // NW

_mesh = plsc.VectorSubcoreMesh(core_axis_name="c", subcore_axis_name="s")


@functools.partial(
    pl.kernel,
    mesh=_mesh,
    out_type=jax.ShapeDtypeStruct((B, N, N, EMB), jnp.float32),
    scratch_types=[
        pltpu.VMEM((N,), jnp.int32),
        pltpu.VMEM((N, EMB), jnp.float32),
        pltpu.SemaphoreType.DMA,
    ],
    compiler_params=pltpu.CompilerParams(use_tc_tiling_on_sc=False),
)
def _gather_kernel(table_hbm, idx_hbm, out_hbm, idx_v, rows_v, sem):
    wid = lax.axis_index("s") * _NC + lax.axis_index("c")
    base = wid * PER_W

    def step(s, carry):
        p = base + s
        b = p // N
        i = p % N
        pltpu.sync_copy(idx_hbm.at[b, i], idx_v)
        pltpu.async_copy(table_hbm.at[idx_v], rows_v, sem).wait()
        pltpu.sync_copy(rows_v, out_hbm.at[b, i])
        return carry

    lax.fori_loop(0, PER_W, step, 0)


def kernel(spatial_pos, W):
    return _gather_kernel(W, spatial_pos.astype(jnp.int32))


# speedup vs baseline: 11.2622x; 1.5761x over previous
import functools

import jax
import jax.numpy as jnp
from jax import lax
from jax.experimental import pallas as pl
from jax.experimental.pallas import tpu as pltpu
from jax.experimental.pallas import tpu_sc as plsc

B, N, EMB = 8, 512, 16
PAIRS = B * N
_NC, _NS = 2, 16
NW = _NC * _NS
PER_W = PAIRS // NW
ET, JT, EL, JL = 2, 4, 8, 128
TBL = 512 * EMB

_mesh = plsc.VectorSubcoreMesh(core_axis_name="c", subcore_axis_name="s")


@functools.partial(
    pl.kernel,
    mesh=_mesh,
    out_type=jax.ShapeDtypeStruct((B, N, ET * JT * EL * JL), jnp.float32),
    scratch_types=[
        pltpu.VMEM((TBL,), jnp.float32),
        pltpu.VMEM((N,), jnp.int32),
        pltpu.VMEM((ET * JT * EL * JL,), jnp.float32),
    ],
    compiler_params=pltpu.CompilerParams(
        use_tc_tiling_on_sc=False, needs_layout_passes=False
    ),
)
def _gather_kernel(table_hbm, idx_hbm, out_hbm, table_v, idx_v, rowsT):
    wid = lax.axis_index("s") * _NC + lax.axis_index("c")
    pltpu.sync_copy(table_hbm, table_v)

    def step(s, carry):
        p = wid * PER_W + s
        b = p // N
        i = p % N
        pltpu.sync_copy(idx_hbm.at[b, i], idx_v)
        for jb in range(N // 16):
            idxv = idx_v[pl.ds(16 * jb, 16)]
            fv = idxv * EMB
            jt = (jb * 16) // JL
            lo = (jb * 16) % JL
            for e in range(EMB):
                g = plsc.load_gather(table_v, [fv + e])
                off = ((e // EL) * JT + jt) * (EL * JL) + (e % EL) * JL + lo
                rowsT[pl.ds(off, 16)] = g
        pltpu.sync_copy(rowsT, out_hbm.at[b, i])
        return carry

    lax.fori_loop(0, PER_W, step, 0)


def kernel(spatial_pos, W):
    out3 = _gather_kernel(W.reshape(TBL), spatial_pos.astype(jnp.int32))
    out6 = out3.reshape(B, N, ET, JT, EL, JL)
    t = out6.transpose(0, 1, 3, 5, 2, 4)
    return t.reshape(B, N, N, EMB)


# speedup vs baseline: 12.8258x; 1.1388x over previous
import functools

import jax
import jax.numpy as jnp
from jax import lax
from jax.experimental import pallas as pl
from jax.experimental.pallas import tpu as pltpu
from jax.experimental.pallas import tpu_sc as plsc

B, N, EMB = 8, 512, 16
PAIRS = B * N
_NC, _NS = 2, 16
NW = _NC * _NS
PER_W = PAIRS // NW
ET, JT, EL, JL = 2, 4, 8, 128
BLK = ET * JT * EL * JL
TBL = 512 * EMB

_mesh = plsc.VectorSubcoreMesh(core_axis_name="c", subcore_axis_name="s")


@functools.partial(
    pl.kernel,
    mesh=_mesh,
    out_type=jax.ShapeDtypeStruct((B, N, BLK), jnp.float32),
    scratch_types=[
        pltpu.VMEM((TBL,), jnp.float32),
        pltpu.VMEM((2 * N,), jnp.int32),
        pltpu.VMEM((2 * BLK,), jnp.float32),
        pltpu.SemaphoreType.DMA,
        pltpu.SemaphoreType.DMA,
        pltpu.SemaphoreType.DMA,
        pltpu.SemaphoreType.DMA,
    ],
    compiler_params=pltpu.CompilerParams(
        use_tc_tiling_on_sc=False, needs_layout_passes=False
    ),
)
def _gather_kernel(table_hbm, idx_hbm, out_hbm, table_v, idx_v, rowsT,
                   sem_i0, sem_i1, sem_w0, sem_w1):
    wid = lax.axis_index("s") * _NC + lax.axis_index("c")
    base = wid * PER_W
    pltpu.sync_copy(table_hbm, table_v)
    sem_i = (sem_i0, sem_i1)
    sem_w = (sem_w0, sem_w1)

    def idx_src(p):
        return idx_hbm.at[p // N, p % N]

    pltpu.async_copy(idx_src(base), idx_v.at[pl.ds(0, N)], sem_i[0])

    def pair(g, carry):
        for k in (0, 1):
            s = 2 * g + k
            p = base + s
            b = p // N
            i = p % N
            pltpu.make_async_copy(
                idx_src(p), idx_v.at[pl.ds(k * N, N)], sem_i[k]).wait()
            @pl.when(s + 1 < PER_W)
            def _():
                pltpu.async_copy(
                    idx_src(p + 1), idx_v.at[pl.ds((1 - k) * N, N)],
                    sem_i[1 - k])
            @pl.when(s >= 2)
            def _():
                pltpu.make_async_copy(
                    rowsT.at[pl.ds(k * BLK, BLK)], out_hbm.at[b, i],
                    sem_w[k]).wait()
            kb = k * BLK
            kn = k * N
            for jb in range(N // 16):
                idxv = idx_v[pl.ds(kn + 16 * jb, 16)]
                fv = idxv * EMB
                jt = (jb * 16) // JL
                lo = (jb * 16) % JL
                for e in range(EMB):
                    g16 = plsc.load_gather(table_v, [fv + e])
                    off = kb + ((e // EL) * JT + jt) * (EL * JL) + (e % EL) * JL + lo
                    rowsT[pl.ds(off, 16)] = g16
            pltpu.async_copy(
                rowsT.at[pl.ds(kb, BLK)], out_hbm.at[b, i], sem_w[k])
        return carry

    lax.fori_loop(0, PER_W // 2, pair, 0)
    for k in (0, 1):
        p = base + PER_W - 2 + k
        pltpu.make_async_copy(
            rowsT.at[pl.ds(k * BLK, BLK)], out_hbm.at[p // N, p % N],
            sem_w[k]).wait()


def kernel(spatial_pos, W):
    out3 = _gather_kernel(W.reshape(TBL), spatial_pos.astype(jnp.int32))
    out6 = out3.reshape(B, N, ET, JT, EL, JL)
    t = out6.transpose(0, 1, 3, 5, 2, 4)
    return t.reshape(B, N, N, EMB)


# speedup vs baseline: 37.6331x; 2.9342x over previous
import functools

import jax
import jax.numpy as jnp
from jax import lax
from jax.experimental import pallas as pl
from jax.experimental.pallas import tpu as pltpu
from jax.experimental.pallas import tpu_sc as plsc

B, N, EMB = 8, 512, 16
PAIRS = B * N
_NC, _NS = 2, 16
NW = _NC * _NS
PER_W = PAIRS // NW
ET, JT, EL, JL = 2, 4, 8, 128
BLK = ET * JT * EL * JL
TBL = 512 * EMB

_mesh = plsc.VectorSubcoreMesh(core_axis_name="c", subcore_axis_name="s")


@functools.partial(
    pl.kernel,
    mesh=_mesh,
    out_type=jax.ShapeDtypeStruct((B, N, BLK), jnp.float32),
    scratch_types=[
        pltpu.VMEM((TBL,), jnp.float32),
        pltpu.VMEM((2 * N,), jnp.int32),
        pltpu.VMEM((2 * BLK,), jnp.float32),
        pltpu.SemaphoreType.DMA,
        pltpu.SemaphoreType.DMA,
        pltpu.SemaphoreType.DMA,
        pltpu.SemaphoreType.DMA,
    ],
    compiler_params=pltpu.CompilerParams(
        use_tc_tiling_on_sc=False, needs_layout_passes=False
    ),
)
def _gather_kernel(table_hbm, idx_hbm, out_hbm, table_v, idx_v, rowsT,
                   sem_i0, sem_i1, sem_w0, sem_w1):
    wid = lax.axis_index("s") * _NC + lax.axis_index("c")
    base = wid * PER_W
    pltpu.sync_copy(table_hbm, table_v)
    sem_i = (sem_i0, sem_i1)
    sem_w = (sem_w0, sem_w1)

    def idx_src(p):
        return idx_hbm.at[p // N, p % N]

    pltpu.async_copy(idx_src(base), idx_v.at[pl.ds(0, N)], sem_i[0])

    def pair(g, carry):
        for k in (0, 1):
            s = 2 * g + k
            p = base + s
            b = p // N
            i = p % N
            pltpu.make_async_copy(
                idx_src(p), idx_v.at[pl.ds(k * N, N)], sem_i[k]).wait()
            @pl.when(s + 1 < PER_W)
            def _():
                pltpu.async_copy(
                    idx_src(p + 1), idx_v.at[pl.ds((1 - k) * N, N)],
                    sem_i[1 - k])
            @pl.when(s >= 2)
            def _():
                pltpu.make_async_copy(
                    rowsT.at[pl.ds(k * BLK, BLK)], out_hbm.at[b, i],
                    sem_w[k]).wait()
            kb = k * BLK
            kn = k * N

            @plsc.parallel_loop(0, N // 16, unroll=2)
            def _jb(jb):
                idxv = idx_v[pl.ds(kn + 16 * jb, 16)]
                fv = idxv * EMB
                col = jb * 16
                tilebase = (col // JL) * (EL * JL) + col % JL
                gs = [plsc.load_gather(table_v, [fv + e]) for e in range(EMB)]
                for e in range(EMB):
                    off = kb + (e // EL) * (JT * EL * JL) + (e % EL) * JL
                    rowsT[pl.ds(off + tilebase, 16)] = gs[e]
            pltpu.async_copy(
                rowsT.at[pl.ds(kb, BLK)], out_hbm.at[b, i], sem_w[k])
        return carry

    lax.fori_loop(0, PER_W // 2, pair, 0)
    for k in (0, 1):
        p = base + PER_W - 2 + k
        pltpu.make_async_copy(
            rowsT.at[pl.ds(k * BLK, BLK)], out_hbm.at[p // N, p % N],
            sem_w[k]).wait()


def kernel(spatial_pos, W):
    out3 = _gather_kernel(W.reshape(TBL), spatial_pos.astype(jnp.int32))
    out6 = out3.reshape(B, N, ET, JT, EL, JL)
    t = out6.transpose(0, 1, 3, 5, 2, 4)
    return t.reshape(B, N, N, EMB)
